# baseline (speedup 1.0000x reference)
import functools

import jax
import jax.numpy as jnp
from jax import lax
from jax.experimental import pallas as pl
from jax.experimental.pallas import tpu as pltpu
from jax.experimental.pallas import tpu_sc as plsc

NN = 10000
EE = 160000
CIN = 128
HH = 32

NC = 2
NS = 16
NWORK = NC * NS

NPAD = 10240
RPT = NPAD // NS
CH = 128
NCHK = 40
EPT = CH * NCHK
EPAD = EPT * NWORK
BR = 400

def _sc_mesh():
    return plsc.VectorSubcoreMesh(core_axis_name="c", subcore_axis_name="s")


_SC_PARAMS = pltpu.CompilerParams(use_tc_tiling_on_sc=False,
                                  needs_layout_passes=False)


def _wid():
    return lax.axis_index("s") * NC + lax.axis_index("c")


def _deg_body(dstT, dstL, degT_out, degL_out, idx_v, ones_v, zeros_v,
              accT, accL, sem):
    c = lax.axis_index("c")
    s = lax.axis_index("s")
    w = _wid()

    for i in range(RPT // 16):
        zeros_v[pl.ds(i * 16, 16)] = jnp.zeros((16,), jnp.float32)
    for i in range(CH // 16):
        ones_v[pl.ds(i * 16, 16)] = jnp.ones((16,), jnp.float32)
    pltpu.sync_copy(zeros_v, accT.at[pl.ds(s * RPT, RPT)])
    pltpu.sync_copy(zeros_v, accL.at[pl.ds(s * RPT, RPT)])
    plsc.subcore_barrier()

    pltpu.sync_copy(dstT.at[w], idx_v)

    def hist_t(ch, _):
        pltpu.sync_copy(ones_v, accT.at[idx_v.at[ch]], add=True)
        return 0
    lax.fori_loop(0, NCHK, hist_t, 0)
    pltpu.sync_copy(dstL.at[w], idx_v)

    def hist_l(ch, _):
        pltpu.sync_copy(ones_v, accL.at[idx_v.at[ch]], add=True)
        return 0
    lax.fori_loop(0, NCHK, hist_l, 0)
    plsc.subcore_barrier()

    pltpu.sync_copy(accT.at[pl.ds(s * RPT, RPT)], degT_out.at[c, pl.ds(s * RPT, RPT)])
    pltpu.sync_copy(accL.at[pl.ds(s * RPT, RPT)], degL_out.at[c, pl.ds(s * RPT, RPT)])


@functools.cache
def _deg_kernel():
    return pl.kernel(
        _deg_body,
        out_type=[jax.ShapeDtypeStruct((NC, NPAD), jnp.float32),
                  jax.ShapeDtypeStruct((NC, NPAD), jnp.float32)],
        mesh=_sc_mesh(),
        scratch_types=[
        pltpu.VMEM((NCHK, CH), jnp.int32),
        pltpu.VMEM((CH,), jnp.float32),
        pltpu.VMEM((RPT,), jnp.float32),
            pltpu.VMEM_SHARED((NPAD,), jnp.float32),
            pltpu.VMEM_SHARED((NPAD,), jnp.float32),
            pltpu.SemaphoreType.DMA,
        ],
        compiler_params=_SC_PARAMS,
    )


def _enc_body(x_ref, w_ref, dT_ref, dL_ref, yT_ref, yL_ref, vT_ref, vL_ref):
    xb = x_ref[...]
    ss = jnp.sum(xb * xb, axis=1, keepdims=True)
    xn = xb / jnp.clip(jnp.sqrt(ss), 1e-12, None)
    xw = jnp.dot(xn, w_ref[...], preferred_element_type=jnp.float32)
    vT = lax.rsqrt(dT_ref[0] + dT_ref[1] + 1.0)
    vL = lax.rsqrt(dL_ref[0] + dL_ref[1] + 1.0)
    yT_ref[...] = xw * vT
    yL_ref[...] = xw * vL
    vT_ref[...] = vT
    vL_ref[...] = vL


def _encode(x, W1, degT, degL):
    return pl.pallas_call(
        _enc_body,
        grid=(NN // BR,),
        in_specs=[
            pl.BlockSpec((BR, CIN), lambda i: (i, 0)),
            pl.BlockSpec((CIN, HH), lambda i: (0, 0)),
            pl.BlockSpec((NC, BR, 1), lambda i: (0, i, 0)),
            pl.BlockSpec((NC, BR, 1), lambda i: (0, i, 0)),
        ],
        out_specs=[
            pl.BlockSpec((BR, HH), lambda i: (i, 0)),
            pl.BlockSpec((BR, HH), lambda i: (i, 0)),
            pl.BlockSpec((BR, 1), lambda i: (i, 0)),
            pl.BlockSpec((BR, 1), lambda i: (i, 0)),
        ],
        out_shape=[jax.ShapeDtypeStruct((NPAD, HH), jnp.float32),
                   jax.ShapeDtypeStruct((NPAD, HH), jnp.float32),
                   jax.ShapeDtypeStruct((NPAD, 1), jnp.float32),
                   jax.ShapeDtypeStruct((NPAD, 1), jnp.float32)],
    )(x, W1, degT.reshape(NC, NPAD, 1), degL.reshape(NC, NPAD, 1))


def _scat_body(yT, yL, srcT, dstT, srcL, dstL, accT_out, accL_out,
               isrc, idst, rows0, rows1, zbuf, accT, accL,
               sem0, sem1):
    c = lax.axis_index("c")
    s = lax.axis_index("s")
    w = _wid()
    rsl = pl.ds(s * RPT, RPT)

    def zrow(i, _):
        zbuf[i, pl.ds(0, 16)] = jnp.zeros((16,), jnp.float32)
        zbuf[i, pl.ds(16, 16)] = jnp.zeros((16,), jnp.float32)
        return 0
    lax.fori_loop(0, RPT, zrow, 0)
    pltpu.sync_copy(zbuf, accT.at[rsl])
    pltpu.sync_copy(zbuf, accL.at[rsl])
    plsc.subcore_barrier()

    for y_s, sidx, didx, acc in ((yT, srcT, dstT, accT),
                                 (yL, srcL, dstL, accL)):
        pltpu.sync_copy(sidx.at[w], isrc)
        pltpu.sync_copy(didx.at[w], idst)
        pltpu.async_copy(y_s.at[isrc.at[0]], rows0, sem0)

        def pair(pp, _, y_s=y_s, acc=acc):
            ch0 = 2 * pp
            pltpu.async_copy(y_s.at[isrc.at[ch0 + 1]], rows1, sem1)
            pltpu.make_async_copy(y_s.at[isrc.at[ch0]], rows0, sem0).wait()
            pltpu.sync_copy(rows0, acc.at[idst.at[ch0]], add=True)

            @pl.when(pp + 1 < NCHK // 2)
            def _():
                pltpu.async_copy(y_s.at[isrc.at[ch0 + 2]], rows0, sem0)
            pltpu.make_async_copy(y_s.at[isrc.at[ch0 + 1]], rows1, sem1).wait()
            pltpu.sync_copy(rows1, acc.at[idst.at[ch0 + 1]], add=True)
            return 0
        lax.fori_loop(0, NCHK // 2, pair, 0)
    plsc.subcore_barrier()

    pltpu.sync_copy(accT.at[pl.ds(s * RPT, RPT)], accT_out.at[c, pl.ds(s * RPT, RPT)])
    pltpu.sync_copy(accL.at[pl.ds(s * RPT, RPT)], accL_out.at[c, pl.ds(s * RPT, RPT)])


@functools.cache
def _scat_kernel():
    return pl.kernel(
        _scat_body,
        out_type=[jax.ShapeDtypeStruct((NC, NPAD, HH), jnp.float32),
                  jax.ShapeDtypeStruct((NC, NPAD, HH), jnp.float32)],
        mesh=_sc_mesh(),
        scratch_types=[
            pltpu.VMEM((NCHK, CH), jnp.int32),
            pltpu.VMEM((NCHK, CH), jnp.int32),
            pltpu.VMEM((CH, HH), jnp.float32),
            pltpu.VMEM((CH, HH), jnp.float32),
            pltpu.VMEM((RPT, HH), jnp.float32),
            pltpu.VMEM_SHARED((NPAD, HH), jnp.float32),
            pltpu.VMEM_SHARED((NPAD, HH), jnp.float32),
            pltpu.SemaphoreType.DMA,
            pltpu.SemaphoreType.DMA,
        ],
        compiler_params=_SC_PARAMS,
    )


def _fin_body(yT, aT, vT, yL, aL, vL, b, zt, zl):
    zt[...] = vT[...] * (yT[...] + aT[0] + aT[1]) + b[...]
    zl[...] = vL[...] * (yL[...] + aL[0] + aL[1]) + b[...]


def _finalize(yT, accT, vT, yL, accL, vL, b1):
    return pl.pallas_call(
        _fin_body,
        grid=(NN // BR,),
        in_specs=[
            pl.BlockSpec((BR, HH), lambda i: (i, 0)),
            pl.BlockSpec((NC, BR, HH), lambda i: (0, i, 0)),
            pl.BlockSpec((BR, 1), lambda i: (i, 0)),
            pl.BlockSpec((BR, HH), lambda i: (i, 0)),
            pl.BlockSpec((NC, BR, HH), lambda i: (0, i, 0)),
            pl.BlockSpec((BR, 1), lambda i: (i, 0)),
            pl.BlockSpec((1, HH), lambda i: (0, 0)),
        ],
        out_specs=[
            pl.BlockSpec((BR, HH), lambda i: (i, 0)),
            pl.BlockSpec((BR, HH), lambda i: (i, 0)),
        ],
        out_shape=[jax.ShapeDtypeStruct((NN, HH), jnp.float32),
                   jax.ShapeDtypeStruct((NN, HH), jnp.float32)],
    )(yT, accT, vT, yL, accL, vL, b1.reshape(1, HH))


def _edge_body(ztop, e0, e1, ae_out, i0, i1,
               rA0, rB0, rA1, rB1, rA2, rB2, rA3, rB3, dots,
               sem0, sem1, sem2, sem3):
    w = _wid()
    pltpu.sync_copy(e0.at[w], i0)
    pltpu.sync_copy(e1.at[w], i1)

    bufs = ((rA0, rB0, sem0), (rA1, rB1, sem1),
            (rA2, rB2, sem2), (rA3, rB3, sem3))

    def issue(ch, rA, rB, sem):
        pltpu.async_copy(ztop.at[i0.at[ch]], rA, sem)
        pltpu.async_copy(ztop.at[i1.at[ch]], rB, sem)

    def drain(ch, rA, rB, sem):
        pltpu.make_async_copy(ztop.at[i0.at[ch]], rA, sem).wait()
        pltpu.make_async_copy(ztop.at[i1.at[ch]], rB, sem).wait()

    lanes = lax.iota(jnp.int32, 16)

    def compute(ch, rA, rB):
        for g in range(CH // 16):
            rows = lanes + g * 16
            acc = [jnp.zeros((16,), jnp.float32) for _ in range(4)]
            for k in range(HH):
                cols = (lanes + k) & (HH - 1)
                acc[k % 4] = acc[k % 4] + (plsc.load_gather(rA, [rows, cols])
                                           * plsc.load_gather(rB, [rows, cols]))
            dots[ch, pl.ds(g * 16, 16)] = (acc[0] + acc[1]) + (acc[2] + acc[3])

    for j in range(3):
        issue(j, *bufs[j])

    def quad(qq, _):
        base = 4 * qq
        for j in range(4):
            ch = base + j
            nxt = bufs[(j + 3) % 4]

            @pl.when(ch + 3 < NCHK)
            def _(ch=ch, nxt=nxt):
                issue(ch + 3, *nxt)
            drain(ch, *bufs[j])
            compute(ch, bufs[j][0], bufs[j][1])
        return 0
    lax.fori_loop(0, NCHK // 4, quad, 0)

    def sig(g, _):
        row = g // (CH // 16)
        col = (g % (CH // 16)) * 16
        v = dots[row, pl.ds(col, 16)]
        dots[row, pl.ds(col, 16)] = 1.0 / (1.0 + jnp.exp(-v))
        return 0
    lax.fori_loop(0, NCHK * (CH // 16), sig, 0)
    pltpu.sync_copy(dots, ae_out.at[w])


@functools.cache
def _edge_kernel():
    return pl.kernel(
        _edge_body,
        out_type=jax.ShapeDtypeStruct((NWORK, NCHK, CH), jnp.float32),
        mesh=_sc_mesh(),
        scratch_types=(
            [pltpu.VMEM((NCHK, CH), jnp.int32)] * 2
            + [pltpu.VMEM((CH, HH), jnp.float32)] * 8
            + [pltpu.VMEM((NCHK, CH), jnp.float32)]
            + [pltpu.SemaphoreType.DMA] * 4
        ),
        compiler_params=_SC_PARAMS,
    )


def _pad_idx(idx, fill):
    p = jnp.full((EPAD,), fill, jnp.int32).at[:EE].set(idx.astype(jnp.int32))
    return p.reshape(NWORK, NCHK, CH)


def kernel(x, edge_index, lastedg_index, W1, b1):
    srcT = _pad_idx(edge_index[0], 0)
    dstT = _pad_idx(edge_index[1], NN)
    dstT0 = _pad_idx(edge_index[1], 0)
    srcL = _pad_idx(lastedg_index[0], 0)
    dstL = _pad_idx(lastedg_index[1], NN)

    degT, degL = _deg_kernel()(dstT, dstL)
    yT, yL, vT, vL = _encode(x, W1, degT, degL)
    accT, accL = _scat_kernel()(yT, yL, srcT, dstT, srcL, dstL)
    zt, zl = _finalize(yT, accT, vT, yL, accL, vL, b1)
    ae = _edge_kernel()(zt, srcT, dstT0)

    return (zt, zl, ae.reshape(-1)[:EE])


# speedup vs baseline: 1.5968x; 1.5968x over previous
import functools

import jax
import jax.numpy as jnp
from jax import lax
from jax.experimental import pallas as pl
from jax.experimental.pallas import tpu as pltpu
from jax.experimental.pallas import tpu_sc as plsc

NN = 10000
EE = 160000
CIN = 128
HH = 32

NC = 2
NS = 16
NWORK = NC * NS

NPAD = 10240
RPT = NPAD // NS
CH = 128
NCHK = 40
EPT = CH * NCHK
EPAD = EPT * NWORK
BR = 1024

def _sc_mesh():
    return plsc.VectorSubcoreMesh(core_axis_name="c", subcore_axis_name="s")


_SC_PARAMS = pltpu.CompilerParams(use_tc_tiling_on_sc=False,
                                  needs_layout_passes=False)


def _wid():
    return lax.axis_index("s") * NC + lax.axis_index("c")


def _deg_body(dstT, dstL, degT_out, degL_out, idx_v, ones_v, zeros_v,
              accT, accL, sem):
    c = lax.axis_index("c")
    s = lax.axis_index("s")
    w = _wid()

    for i in range(RPT // 16):
        zeros_v[pl.ds(i * 16, 16)] = jnp.zeros((16,), jnp.float32)
    for i in range(CH // 16):
        ones_v[pl.ds(i * 16, 16)] = jnp.ones((16,), jnp.float32)
    pltpu.sync_copy(zeros_v, accT.at[pl.ds(s * RPT, RPT)])
    pltpu.sync_copy(zeros_v, accL.at[pl.ds(s * RPT, RPT)])
    plsc.subcore_barrier()

    pltpu.sync_copy(dstT.at[w], idx_v)

    def hist_t(ch, _):
        pltpu.sync_copy(ones_v, accT.at[idx_v.at[ch]], add=True)
        return 0
    lax.fori_loop(0, NCHK, hist_t, 0)
    pltpu.sync_copy(dstL.at[w], idx_v)

    def hist_l(ch, _):
        pltpu.sync_copy(ones_v, accL.at[idx_v.at[ch]], add=True)
        return 0
    lax.fori_loop(0, NCHK, hist_l, 0)
    plsc.subcore_barrier()

    pltpu.sync_copy(accT.at[pl.ds(s * RPT, RPT)], degT_out.at[c, pl.ds(s * RPT, RPT)])
    pltpu.sync_copy(accL.at[pl.ds(s * RPT, RPT)], degL_out.at[c, pl.ds(s * RPT, RPT)])


@functools.cache
def _deg_kernel():
    return pl.kernel(
        _deg_body,
        out_type=[jax.ShapeDtypeStruct((NC, NPAD), jnp.float32),
                  jax.ShapeDtypeStruct((NC, NPAD), jnp.float32)],
        mesh=_sc_mesh(),
        scratch_types=[
        pltpu.VMEM((NCHK, CH), jnp.int32),
        pltpu.VMEM((CH,), jnp.float32),
        pltpu.VMEM((RPT,), jnp.float32),
            pltpu.VMEM_SHARED((NPAD,), jnp.float32),
            pltpu.VMEM_SHARED((NPAD,), jnp.float32),
            pltpu.SemaphoreType.DMA,
        ],
        compiler_params=_SC_PARAMS,
    )


def _enc_body(x_ref, w_ref, dT_ref, dL_ref, yT_ref, yL_ref, vT_ref, vL_ref):
    xb = x_ref[...]
    ss = jnp.sum(xb * xb, axis=1, keepdims=True)
    xn = xb / jnp.clip(jnp.sqrt(ss), 1e-12, None)
    xw = jnp.dot(xn, w_ref[...], preferred_element_type=jnp.float32)
    vT = lax.rsqrt(dT_ref[0] + dT_ref[1] + 1.0)
    vL = lax.rsqrt(dL_ref[0] + dL_ref[1] + 1.0)
    yT_ref[...] = xw * vT
    yL_ref[...] = xw * vL
    vT_ref[...] = vT
    vL_ref[...] = vL


def _encode(x_p, W1, degT, degL):
    return pl.pallas_call(
        _enc_body,
        grid=(NPAD // BR,),
        in_specs=[
            pl.BlockSpec((BR, CIN), lambda i: (i, 0)),
            pl.BlockSpec((CIN, HH), lambda i: (0, 0)),
            pl.BlockSpec((NC, BR, 1), lambda i: (0, i, 0)),
            pl.BlockSpec((NC, BR, 1), lambda i: (0, i, 0)),
        ],
        out_specs=[
            pl.BlockSpec((BR, HH), lambda i: (i, 0)),
            pl.BlockSpec((BR, HH), lambda i: (i, 0)),
            pl.BlockSpec((BR, 1), lambda i: (i, 0)),
            pl.BlockSpec((BR, 1), lambda i: (i, 0)),
        ],
        out_shape=[jax.ShapeDtypeStruct((NPAD, HH), jnp.float32),
                   jax.ShapeDtypeStruct((NPAD, HH), jnp.float32),
                   jax.ShapeDtypeStruct((NPAD, 1), jnp.float32),
                   jax.ShapeDtypeStruct((NPAD, 1), jnp.float32)],
    )(x_p, W1, degT.reshape(NC, NPAD, 1), degL.reshape(NC, NPAD, 1))


def _scat_body(yT, yL, srcT, dstT, srcL, dstL, accT_out, accL_out,
               isrc, idst, rows0, rows1, zbuf, yTs, yLs, accT, accL,
               sem0, sem1):
    c = lax.axis_index("c")
    s = lax.axis_index("s")
    w = _wid()
    rsl = pl.ds(s * RPT, RPT)

    def zrow(i, _):
        zbuf[i, pl.ds(0, 16)] = jnp.zeros((16,), jnp.float32)
        zbuf[i, pl.ds(16, 16)] = jnp.zeros((16,), jnp.float32)
        return 0
    lax.fori_loop(0, RPT, zrow, 0)
    pltpu.sync_copy(zbuf, accT.at[rsl])
    pltpu.sync_copy(zbuf, accL.at[rsl])
    pltpu.sync_copy(yT.at[rsl], yTs.at[rsl])
    pltpu.sync_copy(yL.at[rsl], yLs.at[rsl])
    plsc.subcore_barrier()

    for y_s, sidx, didx, acc in ((yTs, srcT, dstT, accT),
                                 (yLs, srcL, dstL, accL)):
        pltpu.sync_copy(sidx.at[w], isrc)
        pltpu.sync_copy(didx.at[w], idst)
        pltpu.async_copy(y_s.at[isrc.at[0]], rows0, sem0)

        def pair(pp, _, y_s=y_s, acc=acc):
            ch0 = 2 * pp
            pltpu.async_copy(y_s.at[isrc.at[ch0 + 1]], rows1, sem1)
            pltpu.make_async_copy(y_s.at[isrc.at[ch0]], rows0, sem0).wait()
            pltpu.sync_copy(rows0, acc.at[idst.at[ch0]], add=True)

            @pl.when(pp + 1 < NCHK // 2)
            def _():
                pltpu.async_copy(y_s.at[isrc.at[ch0 + 2]], rows0, sem0)
            pltpu.make_async_copy(y_s.at[isrc.at[ch0 + 1]], rows1, sem1).wait()
            pltpu.sync_copy(rows1, acc.at[idst.at[ch0 + 1]], add=True)
            return 0
        lax.fori_loop(0, NCHK // 2, pair, 0)
    plsc.subcore_barrier()

    pltpu.sync_copy(accT.at[pl.ds(s * RPT, RPT)], accT_out.at[c, pl.ds(s * RPT, RPT)])
    pltpu.sync_copy(accL.at[pl.ds(s * RPT, RPT)], accL_out.at[c, pl.ds(s * RPT, RPT)])


@functools.cache
def _scat_kernel():
    return pl.kernel(
        _scat_body,
        out_type=[jax.ShapeDtypeStruct((NC, NPAD, HH), jnp.float32),
                  jax.ShapeDtypeStruct((NC, NPAD, HH), jnp.float32)],
        mesh=_sc_mesh(),
        scratch_types=[
            pltpu.VMEM((NCHK, CH), jnp.int32),
            pltpu.VMEM((NCHK, CH), jnp.int32),
            pltpu.VMEM((CH, HH), jnp.float32),
            pltpu.VMEM((CH, HH), jnp.float32),
            pltpu.VMEM((RPT, HH), jnp.float32),
            pltpu.VMEM_SHARED((NPAD, HH), jnp.float32),
            pltpu.VMEM_SHARED((NPAD, HH), jnp.float32),
            pltpu.VMEM_SHARED((NPAD, HH), jnp.float32),
            pltpu.VMEM_SHARED((NPAD, HH), jnp.float32),
            pltpu.SemaphoreType.DMA,
            pltpu.SemaphoreType.DMA,
        ],
        compiler_params=_SC_PARAMS,
    )


def _fin_body(yT, aT, vT, yL, aL, vL, b, zt, zl):
    zt[...] = vT[...] * (yT[...] + aT[0] + aT[1]) + b[...]
    zl[...] = vL[...] * (yL[...] + aL[0] + aL[1]) + b[...]


def _finalize(yT, accT, vT, yL, accL, vL, b1):
    return pl.pallas_call(
        _fin_body,
        grid=(NPAD // BR,),
        in_specs=[
            pl.BlockSpec((BR, HH), lambda i: (i, 0)),
            pl.BlockSpec((NC, BR, HH), lambda i: (0, i, 0)),
            pl.BlockSpec((BR, 1), lambda i: (i, 0)),
            pl.BlockSpec((BR, HH), lambda i: (i, 0)),
            pl.BlockSpec((NC, BR, HH), lambda i: (0, i, 0)),
            pl.BlockSpec((BR, 1), lambda i: (i, 0)),
            pl.BlockSpec((1, HH), lambda i: (0, 0)),
        ],
        out_specs=[
            pl.BlockSpec((BR, HH), lambda i: (i, 0)),
            pl.BlockSpec((BR, HH), lambda i: (i, 0)),
        ],
        out_shape=[jax.ShapeDtypeStruct((NPAD, HH), jnp.float32),
                   jax.ShapeDtypeStruct((NPAD, HH), jnp.float32)],
    )(yT, accT, vT, yL, accL, vL, b1.reshape(1, HH))


def _edge_body(ztop, e0, e1, ae_out, i0, i1,
               rA0, rB0, rA1, rB1, rA2, rB2, rA3, rB3, dots, zS,
               sem0, sem1, sem2, sem3):
    s = lax.axis_index("s")
    w = _wid()
    rsl = pl.ds(s * RPT, RPT)
    pltpu.sync_copy(e0.at[w], i0)
    pltpu.sync_copy(e1.at[w], i1)
    pltpu.sync_copy(ztop.at[rsl], zS.at[rsl])
    plsc.subcore_barrier()

    bufs = ((rA0, rB0, sem0), (rA1, rB1, sem1),
            (rA2, rB2, sem2), (rA3, rB3, sem3))

    def issue(ch, rA, rB, sem):
        pltpu.async_copy(zS.at[i0.at[ch]], rA, sem)
        pltpu.async_copy(zS.at[i1.at[ch]], rB, sem)

    def drain(ch, rA, rB, sem):
        pltpu.make_async_copy(zS.at[i0.at[ch]], rA, sem).wait()
        pltpu.make_async_copy(zS.at[i1.at[ch]], rB, sem).wait()

    lanes = lax.iota(jnp.int32, 16)

    def compute(ch, rA, rB):
        for g in range(CH // 16):
            rows = lanes + g * 16
            acc = [jnp.zeros((16,), jnp.float32) for _ in range(4)]
            for k in range(HH):
                cols = (lanes + k) & (HH - 1)
                acc[k % 4] = acc[k % 4] + (plsc.load_gather(rA, [rows, cols])
                                           * plsc.load_gather(rB, [rows, cols]))
            dots[ch, pl.ds(g * 16, 16)] = (acc[0] + acc[1]) + (acc[2] + acc[3])

    for j in range(3):
        issue(j, *bufs[j])

    def quad(qq, _):
        base = 4 * qq
        for j in range(4):
            ch = base + j
            nxt = bufs[(j + 3) % 4]

            @pl.when(ch + 3 < NCHK)
            def _(ch=ch, nxt=nxt):
                issue(ch + 3, *nxt)
            drain(ch, *bufs[j])
            compute(ch, bufs[j][0], bufs[j][1])
        return 0
    lax.fori_loop(0, NCHK // 4, quad, 0)

    def sig(g, _):
        row = g // (CH // 16)
        col = (g % (CH // 16)) * 16
        v = dots[row, pl.ds(col, 16)]
        dots[row, pl.ds(col, 16)] = 1.0 / (1.0 + jnp.exp(-v))
        return 0
    lax.fori_loop(0, NCHK * (CH // 16), sig, 0)
    pltpu.sync_copy(dots, ae_out.at[w])


@functools.cache
def _edge_kernel():
    return pl.kernel(
        _edge_body,
        out_type=jax.ShapeDtypeStruct((NWORK, NCHK, CH), jnp.float32),
        mesh=_sc_mesh(),
        scratch_types=(
            [pltpu.VMEM((NCHK, CH), jnp.int32)] * 2
            + [pltpu.VMEM((CH, HH), jnp.float32)] * 8
            + [pltpu.VMEM((NCHK, CH), jnp.float32),
               pltpu.VMEM_SHARED((NPAD, HH), jnp.float32)]
            + [pltpu.SemaphoreType.DMA] * 4
        ),
        compiler_params=_SC_PARAMS,
    )


def _pad_idx(idx, fill):
    p = jnp.full((EPAD,), fill, jnp.int32).at[:EE].set(idx.astype(jnp.int32))
    return p.reshape(NWORK, NCHK, CH)


def kernel(x, edge_index, lastedg_index, W1, b1):
    x_p = jnp.zeros((NPAD, CIN), jnp.float32).at[:NN].set(x)
    srcT = _pad_idx(edge_index[0], 0)
    dstT = _pad_idx(edge_index[1], NN)
    srcL = _pad_idx(lastedg_index[0], 0)
    dstL = _pad_idx(lastedg_index[1], NN)

    degT, degL = _deg_kernel()(dstT, dstL)
    yT, yL, vT, vL = _encode(x_p, W1, degT, degL)
    accT, accL = _scat_kernel()(yT, yL, srcT, dstT, srcL, dstL)
    zt, zl = _finalize(yT, accT, vT, yL, accL, vL, b1)
    ae = _edge_kernel()(zt, srcT, dstT)

    return (zt[:NN], zl[:NN], ae.reshape(-1)[:EE])
